# Initial kernel scaffold; baseline (speedup 1.0000x reference)
import jax
import jax.numpy as jnp
from jax.experimental import pallas as pl


def kernel(features, edge_index, W_self0, W_neigh0, b0, W_self1, W_neigh1, b1):
    raise NotImplementedError("write your pallas kernel here")


# speedup vs baseline: 3.3325x; 3.3325x over previous
import jax
import jax.numpy as jnp
from jax import lax
from jax.experimental import pallas as pl
from jax.experimental.pallas import tpu as pltpu
from jax.experimental.pallas import tpu_sc as plsc

N_NODES = 10000
N_EDGES = 320000

NC = 2
NS = 16
NW = NC * NS

CHUNK = 128
CPT = 80
E_PAD = NW * CPT * CHUNK
ROWS_PAD = 10112
RPT = ROWS_PAD // NS


def _sc_rows(d_row):
    mesh = plsc.VectorSubcoreMesh(core_axis_name="c", subcore_axis_name="s",
                                  num_cores=NC, num_subcores=NS)
    out_type = jax.ShapeDtypeStruct((NC, ROWS_PAD, d_row), jnp.float32)
    scratch = [
        pltpu.VMEM_SHARED((ROWS_PAD, d_row), jnp.float32),
        pltpu.VMEM((CPT, CHUNK), jnp.int32),
        pltpu.VMEM((CPT, CHUNK), jnp.int32),
        pltpu.VMEM((CHUNK, d_row), jnp.float32),
        pltpu.SemaphoreType.DMA,
    ]

    def body(table, srcm, dstm, zrows, out, agg_sh, src_v, dst_v, rows_v, sem):
        c = lax.axis_index("c")
        s = lax.axis_index("s")
        wid = c * NS + s

        pltpu.sync_copy(zrows.at[pl.ds(s * RPT, RPT)],
                        agg_sh.at[pl.ds(s * RPT, RPT)])
        pltpu.sync_copy(srcm.at[pl.ds(wid * CPT, CPT)], src_v)
        pltpu.sync_copy(dstm.at[pl.ds(wid * CPT, CPT)], dst_v)
        plsc.subcore_barrier()

        def step(j, carry):
            pltpu.async_copy(table.at[src_v.at[j]], rows_v, sem).wait()
            pltpu.sync_copy(rows_v, agg_sh.at[dst_v.at[j]], add=True)
            return carry

        lax.fori_loop(0, CPT, step, 0)
        plsc.subcore_barrier()

        pltpu.sync_copy(agg_sh.at[pl.ds(s * RPT, RPT)],
                        out.at[c, pl.ds(s * RPT, RPT)])

    return pl.kernel(body, out_type=out_type, mesh=mesh,
                     scratch_types=scratch)


def _sc_deg():
    mesh = plsc.VectorSubcoreMesh(core_axis_name="c", subcore_axis_name="s",
                                  num_cores=NC, num_subcores=NS)
    out_type = jax.ShapeDtypeStruct((NC, ROWS_PAD, 128), jnp.float32)
    scratch = [
        pltpu.VMEM_SHARED((ROWS_PAD, 128), jnp.float32),
        pltpu.VMEM((CPT, CHUNK), jnp.int32),
        pltpu.VMEM((CHUNK, 128), jnp.float32),
    ]

    def body(dstm, zdeg_ones, out, deg_sh, dst_v, ones_v):
        c = lax.axis_index("c")
        s = lax.axis_index("s")
        wid = c * NS + s

        pltpu.sync_copy(zdeg_ones.at[pl.ds(s * RPT, RPT)],
                        deg_sh.at[pl.ds(s * RPT, RPT)])
        pltpu.sync_copy(zdeg_ones.at[pl.ds(ROWS_PAD, CHUNK)], ones_v)
        pltpu.sync_copy(dstm.at[pl.ds(wid * CPT, CPT)], dst_v)
        plsc.subcore_barrier()

        def step(j, carry):
            pltpu.sync_copy(ones_v, deg_sh.at[dst_v.at[j]], add=True)
            return carry

        lax.fori_loop(0, CPT, step, 0)
        plsc.subcore_barrier()

        pltpu.sync_copy(deg_sh.at[pl.ds(s * RPT, RPT)],
                        out.at[c, pl.ds(s * RPT, RPT)])

    return pl.kernel(body, out_type=out_type, mesh=mesh,
                     scratch_types=scratch)


_BLK = 1000


def _mid_body(x, p0a, p0b, da, db, ws0, wn0, b0, h_out):
    agg = p0a[...] + p0b[...]
    inv = 1.0 / jnp.maximum(da[...] + db[...], 1.0)
    hp = jnp.dot(x[...], ws0[...], preferred_element_type=jnp.float32,
                 precision=lax.Precision.HIGHEST)
    hn = jnp.dot(agg * inv, wn0[...], preferred_element_type=jnp.float32,
                 precision=lax.Precision.HIGHEST)
    h_out[...] = jnp.maximum(hp + hn + b0[...], 0.0)


def _final_body(h, p1a, p1b, da, db, ws1, wn1, b1, out):
    agg = p1a[...] + p1b[...]
    inv = 1.0 / jnp.maximum(da[...] + db[...], 1.0)
    sp = jnp.dot(h[...], ws1[...], preferred_element_type=jnp.float32,
                 precision=lax.Precision.HIGHEST)
    sn = jnp.dot(agg * inv, wn1[...], preferred_element_type=jnp.float32,
                 precision=lax.Precision.HIGHEST)
    out[...] = sp + sn + b1[...]


def kernel(features, edge_index, W_self0, W_neigh0, b0, W_self1, W_neigh1, b1):
    n = N_NODES
    src = edge_index[0]
    dst = edge_index[1]
    pad = E_PAD - N_EDGES
    srcm = jnp.concatenate([src, jnp.zeros((pad,), jnp.int32)]).reshape(-1, CHUNK)
    dstm = jnp.concatenate([dst, jnp.full((pad,), n, jnp.int32)]).reshape(-1, CHUNK)
    zrows = jnp.zeros((ROWS_PAD, 128), jnp.float32)
    zdeg_ones = jnp.concatenate(
        [jnp.zeros((ROWS_PAD, 128), jnp.float32),
         jnp.ones((CHUNK, 128), jnp.float32)])

    part0 = _sc_rows(128)(features, srcm, dstm, zrows)
    pdeg = _sc_deg()(dstm, zdeg_ones)

    row_spec = pl.BlockSpec((_BLK, 128), lambda i: (i, 0))
    row64_spec = pl.BlockSpec((_BLK, 64), lambda i: (i, 0))
    deg_spec = row_spec
    w_spec = pl.BlockSpec((128, 128), lambda i: (0, 0))
    w64_spec = pl.BlockSpec((128, 64), lambda i: (0, 0))
    b_spec = pl.BlockSpec((1, 128), lambda i: (0, 0))
    b64_spec = pl.BlockSpec((1, 64), lambda i: (0, 0))

    h = pl.pallas_call(
        _mid_body,
        grid=(n // _BLK,),
        in_specs=[row_spec, row_spec, row_spec, deg_spec, deg_spec,
                  w_spec, w_spec, b_spec],
        out_specs=row_spec,
        out_shape=jax.ShapeDtypeStruct((n, 128), jnp.float32),
    )(features, part0[0], part0[1], pdeg[0], pdeg[1],
      W_self0, W_neigh0, b0.reshape(1, 128))

    part1 = _sc_rows(128)(h, srcm, dstm, zrows)

    out = pl.pallas_call(
        _final_body,
        grid=(n // _BLK,),
        in_specs=[row_spec, row_spec, row_spec, deg_spec, deg_spec,
                  w64_spec, w64_spec, b64_spec],
        out_specs=row64_spec,
        out_shape=jax.ShapeDtypeStruct((n, 64), jnp.float32),
    )(h, part1[0], part1[1], pdeg[0], pdeg[1],
      W_self1, W_neigh1, b1.reshape(1, 64))
    return out


# speedup vs baseline: 3.7066x; 1.1123x over previous
import jax
import jax.numpy as jnp
from jax import lax
from jax.experimental import pallas as pl
from jax.experimental.pallas import tpu as pltpu
from jax.experimental.pallas import tpu_sc as plsc

N_NODES = 10000
N_EDGES = 320000

NC = 2
NS = 16
NW = NC * NS

CHUNK = 128
CPT = 80
NBUF = 2
QC = CPT // 2
E_PAD = NW * CPT * CHUNK
ROWS_PAD = 10112
RPT = ROWS_PAD // NS


def _sc_rows(d_row):
    mesh = plsc.VectorSubcoreMesh(core_axis_name="c", subcore_axis_name="s",
                                  num_cores=NC, num_subcores=NS)
    out_type = jax.ShapeDtypeStruct((NC, ROWS_PAD, d_row), jnp.float32)
    scratch = [
        pltpu.VMEM_SHARED((ROWS_PAD, d_row), jnp.float32),
        pltpu.VMEM((QC, CHUNK), jnp.int32),
        pltpu.VMEM((QC, CHUNK), jnp.int32),
    ] + [pltpu.VMEM((CHUNK, d_row), jnp.float32) for _ in range(NBUF)] \
      + [pltpu.SemaphoreType.DMA for _ in range(NBUF)]

    def body(table, srcm, dstm, zrows, out, agg_sh, src_v, dst_v, *bufs_sems):
        rows_b = bufs_sems[:NBUF]
        sems = bufs_sems[NBUF:]
        c = lax.axis_index("c")
        s = lax.axis_index("s")
        wid = c * NS + s

        pltpu.sync_copy(zrows.at[pl.ds(s * RPT, RPT)],
                        agg_sh.at[pl.ds(s * RPT, RPT)])
        plsc.subcore_barrier()

        def half(q):
            pltpu.sync_copy(srcm.at[pl.ds(wid * CPT + q * QC, QC)], src_v)
            pltpu.sync_copy(dstm.at[pl.ds(wid * CPT + q * QC, QC)], dst_v)
            for b in range(NBUF):
                pltpu.async_copy(table.at[src_v.at[b]], rows_b[b], sems[b])

            def step(g, carry):
                for b in range(NBUF):
                    j = g * NBUF + b
                    pltpu.make_async_copy(table.at[src_v.at[j]], rows_b[b],
                                          sems[b]).wait()
                    pltpu.sync_copy(rows_b[b], agg_sh.at[dst_v.at[j]],
                                    add=True)

                    @pl.when(j + NBUF < QC)
                    def _():
                        pltpu.async_copy(table.at[src_v.at[j + NBUF]],
                                         rows_b[b], sems[b])
                return carry

            lax.fori_loop(0, QC // NBUF, step, 0)

        half(0)
        half(1)
        plsc.subcore_barrier()

        pltpu.sync_copy(agg_sh.at[pl.ds(s * RPT, RPT)],
                        out.at[c, pl.ds(s * RPT, RPT)])

    return pl.kernel(body, out_type=out_type, mesh=mesh,
                     scratch_types=scratch)


def _sc_deg():
    mesh = plsc.VectorSubcoreMesh(core_axis_name="c", subcore_axis_name="s",
                                  num_cores=NC, num_subcores=NS)
    out_type = jax.ShapeDtypeStruct((NC, ROWS_PAD, 128), jnp.float32)
    scratch = [
        pltpu.VMEM_SHARED((ROWS_PAD, 128), jnp.float32),
        pltpu.VMEM((CPT, CHUNK), jnp.int32),
        pltpu.VMEM((CHUNK, 128), jnp.float32),
    ]

    def body(dstm, zdeg_ones, out, deg_sh, dst_v, ones_v):
        c = lax.axis_index("c")
        s = lax.axis_index("s")
        wid = c * NS + s

        pltpu.sync_copy(zdeg_ones.at[pl.ds(s * RPT, RPT)],
                        deg_sh.at[pl.ds(s * RPT, RPT)])
        pltpu.sync_copy(zdeg_ones.at[pl.ds(ROWS_PAD, CHUNK)], ones_v)
        pltpu.sync_copy(dstm.at[pl.ds(wid * CPT, CPT)], dst_v)
        plsc.subcore_barrier()

        def step(j, carry):
            pltpu.sync_copy(ones_v, deg_sh.at[dst_v.at[j]], add=True)
            return carry

        lax.fori_loop(0, CPT, step, 0)
        plsc.subcore_barrier()

        pltpu.sync_copy(deg_sh.at[pl.ds(s * RPT, RPT)],
                        out.at[c, pl.ds(s * RPT, RPT)])

    return pl.kernel(body, out_type=out_type, mesh=mesh,
                     scratch_types=scratch)


_BLK = 1000


def _mid_body(x, p0a, p0b, da, db, ws0, wn0, b0, h_out):
    agg = p0a[...] + p0b[...]
    inv = 1.0 / jnp.maximum(da[...] + db[...], 1.0)
    hp = jnp.dot(x[...], ws0[...], preferred_element_type=jnp.float32,
                 precision=lax.Precision.HIGHEST)
    hn = jnp.dot(agg * inv, wn0[...], preferred_element_type=jnp.float32,
                 precision=lax.Precision.HIGHEST)
    h_out[...] = jnp.maximum(hp + hn + b0[...], 0.0)


def _final_body(h, p1a, p1b, da, db, ws1, wn1, b1, out):
    agg = p1a[...] + p1b[...]
    inv = 1.0 / jnp.maximum(da[...] + db[...], 1.0)
    sp = jnp.dot(h[...], ws1[...], preferred_element_type=jnp.float32,
                 precision=lax.Precision.HIGHEST)
    sn = jnp.dot(agg * inv, wn1[...], preferred_element_type=jnp.float32,
                 precision=lax.Precision.HIGHEST)
    out[...] = sp + sn + b1[...]


def kernel(features, edge_index, W_self0, W_neigh0, b0, W_self1, W_neigh1, b1):
    n = N_NODES
    src = edge_index[0]
    dst = edge_index[1]
    pad = E_PAD - N_EDGES
    srcm = jnp.concatenate([src, jnp.zeros((pad,), jnp.int32)]).reshape(-1, CHUNK)
    dstm = jnp.concatenate([dst, jnp.full((pad,), n, jnp.int32)]).reshape(-1, CHUNK)
    zrows = jnp.zeros((ROWS_PAD, 128), jnp.float32)
    zdeg_ones = jnp.concatenate(
        [jnp.zeros((ROWS_PAD, 128), jnp.float32),
         jnp.ones((CHUNK, 128), jnp.float32)])

    part0 = _sc_rows(128)(features, srcm, dstm, zrows)
    pdeg = _sc_deg()(dstm, zdeg_ones)

    row_spec = pl.BlockSpec((_BLK, 128), lambda i: (i, 0))
    row64_spec = pl.BlockSpec((_BLK, 64), lambda i: (i, 0))
    deg_spec = row_spec
    w_spec = pl.BlockSpec((128, 128), lambda i: (0, 0))
    w64_spec = pl.BlockSpec((128, 64), lambda i: (0, 0))
    b_spec = pl.BlockSpec((1, 128), lambda i: (0, 0))
    b64_spec = pl.BlockSpec((1, 64), lambda i: (0, 0))

    h = pl.pallas_call(
        _mid_body,
        grid=(n // _BLK,),
        in_specs=[row_spec, row_spec, row_spec, deg_spec, deg_spec,
                  w_spec, w_spec, b_spec],
        out_specs=row_spec,
        out_shape=jax.ShapeDtypeStruct((n, 128), jnp.float32),
    )(features, part0[0], part0[1], pdeg[0], pdeg[1],
      W_self0, W_neigh0, b0.reshape(1, 128))

    part1 = _sc_rows(128)(h, srcm, dstm, zrows)

    out = pl.pallas_call(
        _final_body,
        grid=(n // _BLK,),
        in_specs=[row_spec, row_spec, row_spec, deg_spec, deg_spec,
                  w64_spec, w64_spec, b64_spec],
        out_specs=row64_spec,
        out_shape=jax.ShapeDtypeStruct((n, 64), jnp.float32),
    )(h, part1[0], part1[1], pdeg[0], pdeg[1],
      W_self1, W_neigh1, b1.reshape(1, 64))
    return out


# speedup vs baseline: 9.5848x; 2.5859x over previous
import jax
import jax.numpy as jnp
from jax import lax
from jax.experimental import pallas as pl
from jax.experimental.pallas import tpu as pltpu
from jax.experimental.pallas import tpu_sc as plsc

N_NODES = 10000
N_EDGES = 320000

NC = 2
NS = 16
NW = NC * NS

CHUNK = 125
CPT = 80
NBUF = 2
QC = CPT // 2
ROWS_PAD = 10112
RPT = ROWS_PAD // NS


def _sc_rows(d_row):
    mesh = plsc.VectorSubcoreMesh(core_axis_name="c", subcore_axis_name="s",
                                  num_cores=NC, num_subcores=NS)
    out_type = jax.ShapeDtypeStruct((NC, ROWS_PAD, d_row), jnp.float32)
    scratch = [
        pltpu.VMEM_SHARED((ROWS_PAD, d_row), jnp.float32),
        pltpu.VMEM((QC, CHUNK), jnp.int32),
        pltpu.VMEM((QC, CHUNK), jnp.int32),
    ] + [pltpu.VMEM((CHUNK, d_row), jnp.float32) for _ in range(NBUF)] \
      + [pltpu.SemaphoreType.DMA for _ in range(NBUF)]

    def body(table, srcm, dstm, zrows, out, agg_sh, src_v, dst_v, *bufs_sems):
        rows_b = bufs_sems[:NBUF]
        sems = bufs_sems[NBUF:]
        c = lax.axis_index("c")
        s = lax.axis_index("s")
        wid = c * NS + s

        pltpu.sync_copy(zrows.at[pl.ds(s * RPT, RPT)],
                        agg_sh.at[pl.ds(s * RPT, RPT)])
        plsc.subcore_barrier()

        def half(q):
            pltpu.sync_copy(srcm.at[pl.ds(wid * CPT + q * QC, QC)], src_v)
            pltpu.sync_copy(dstm.at[pl.ds(wid * CPT + q * QC, QC)], dst_v)
            for b in range(NBUF):
                pltpu.async_copy(table.at[src_v.at[b]], rows_b[b], sems[b])

            def step(g, carry):
                for b in range(NBUF):
                    j = g * NBUF + b
                    pltpu.make_async_copy(table.at[src_v.at[j]], rows_b[b],
                                          sems[b]).wait()
                    pltpu.sync_copy(rows_b[b], agg_sh.at[dst_v.at[j]],
                                    add=True)

                    @pl.when(j + NBUF < QC)
                    def _():
                        pltpu.async_copy(table.at[src_v.at[j + NBUF]],
                                         rows_b[b], sems[b])
                return carry

            lax.fori_loop(0, QC // NBUF, step, 0)

        half(0)
        half(1)
        plsc.subcore_barrier()

        pltpu.sync_copy(agg_sh.at[pl.ds(s * RPT, RPT)],
                        out.at[c, pl.ds(s * RPT, RPT)])

    return pl.kernel(body, out_type=out_type, mesh=mesh,
                     scratch_types=scratch)


def _sc_deg():
    mesh = plsc.VectorSubcoreMesh(core_axis_name="c", subcore_axis_name="s",
                                  num_cores=NC, num_subcores=NS)
    out_type = jax.ShapeDtypeStruct((NC, ROWS_PAD, 128), jnp.float32)
    scratch = [
        pltpu.VMEM_SHARED((ROWS_PAD, 128), jnp.float32),
        pltpu.VMEM((CPT, CHUNK), jnp.int32),
        pltpu.VMEM((CHUNK, 128), jnp.float32),
    ]

    def body(dstm, zdeg_ones, out, deg_sh, dst_v, ones_v):
        c = lax.axis_index("c")
        s = lax.axis_index("s")
        wid = c * NS + s

        pltpu.sync_copy(zdeg_ones.at[pl.ds(s * RPT, RPT)],
                        deg_sh.at[pl.ds(s * RPT, RPT)])
        pltpu.sync_copy(zdeg_ones.at[pl.ds(ROWS_PAD, CHUNK)], ones_v)
        pltpu.sync_copy(dstm.at[pl.ds(wid * CPT, CPT)], dst_v)
        plsc.subcore_barrier()

        def step(j, carry):
            pltpu.sync_copy(ones_v, deg_sh.at[dst_v.at[j]], add=True)
            return carry

        lax.fori_loop(0, CPT, step, 0)
        plsc.subcore_barrier()

        pltpu.sync_copy(deg_sh.at[pl.ds(s * RPT, RPT)],
                        out.at[c, pl.ds(s * RPT, RPT)])

    return pl.kernel(body, out_type=out_type, mesh=mesh,
                     scratch_types=scratch)


_BLK = 1000


def _mid_body(x, p0a, p0b, da, db, ws0, wn0, b0, h_out):
    agg = p0a[...] + p0b[...]
    inv = 1.0 / jnp.maximum(da[...] + db[...], 1.0)
    hp = jnp.dot(x[...], ws0[...], preferred_element_type=jnp.float32,
                 precision=lax.Precision.HIGHEST)
    hn = jnp.dot(agg * inv, wn0[...], preferred_element_type=jnp.float32,
                 precision=lax.Precision.HIGHEST)
    h_out[...] = jnp.maximum(hp + hn + b0[...], 0.0)


def _final_body(h, p1a, p1b, da, db, ws1, wn1, b1, out):
    agg = p1a[...] + p1b[...]
    inv = 1.0 / jnp.maximum(da[...] + db[...], 1.0)
    sp = jnp.dot(h[...], ws1[...], preferred_element_type=jnp.float32,
                 precision=lax.Precision.HIGHEST)
    sn = jnp.dot(agg * inv, wn1[...], preferred_element_type=jnp.float32,
                 precision=lax.Precision.HIGHEST)
    out[...] = sp + sn + b1[...]


def kernel(features, edge_index, W_self0, W_neigh0, b0, W_self1, W_neigh1, b1):
    n = N_NODES
    srcm = edge_index[0].reshape(-1, CHUNK)
    dstm = edge_index[1].reshape(-1, CHUNK)
    zrows = jnp.zeros((ROWS_PAD, 128), jnp.float32)
    zdeg_ones = jnp.concatenate(
        [jnp.zeros((ROWS_PAD, 128), jnp.float32),
         jnp.ones((CHUNK, 128), jnp.float32)])

    part0 = _sc_rows(128)(features, srcm, dstm, zrows)
    pdeg = _sc_deg()(dstm, zdeg_ones)

    row_spec = pl.BlockSpec((_BLK, 128), lambda i: (i, 0))
    row64_spec = pl.BlockSpec((_BLK, 64), lambda i: (i, 0))
    deg_spec = row_spec
    w_spec = pl.BlockSpec((128, 128), lambda i: (0, 0))
    w64_spec = pl.BlockSpec((128, 64), lambda i: (0, 0))
    b_spec = pl.BlockSpec((1, 128), lambda i: (0, 0))
    b64_spec = pl.BlockSpec((1, 64), lambda i: (0, 0))

    h = pl.pallas_call(
        _mid_body,
        grid=(n // _BLK,),
        in_specs=[row_spec, row_spec, row_spec, deg_spec, deg_spec,
                  w_spec, w_spec, b_spec],
        out_specs=row_spec,
        out_shape=jax.ShapeDtypeStruct((n, 128), jnp.float32),
    )(features, part0[0], part0[1], pdeg[0], pdeg[1],
      W_self0, W_neigh0, b0.reshape(1, 128))

    part1 = _sc_rows(128)(h, srcm, dstm, zrows)

    out = pl.pallas_call(
        _final_body,
        grid=(n // _BLK,),
        in_specs=[row_spec, row_spec, row_spec, deg_spec, deg_spec,
                  w64_spec, w64_spec, b64_spec],
        out_specs=row64_spec,
        out_shape=jax.ShapeDtypeStruct((n, 64), jnp.float32),
    )(h, part1[0], part1[1], pdeg[0], pdeg[1],
      W_self1, W_neigh1, b1.reshape(1, 64))
    return out
